# baseline (speedup 1.0000x reference)
import functools

import jax
import jax.numpy as jnp
from jax import lax
from jax.experimental import pallas as pl
from jax.experimental.pallas import tpu as pltpu
from jax.experimental.pallas import tpu_sc as plsc

N = 10000
E = 320000
D = 128

NC = 2
NS = 16
NW = NC * NS

K = 128
NCHUNK = 2560
CPT = NCHUNK // NW
EPAD = NCHUNK * K - E
NP = 10240
PAD_NODE = N + 100
RPS = NP // NS

_mesh = plsc.VectorSubcoreMesh(core_axis_name="c", subcore_axis_name="s")


@functools.partial(
    pl.kernel,
    out_type=jax.ShapeDtypeStruct((NC, NP, D), jnp.float32),
    mesh=_mesh,
    scratch_types=[
        pltpu.VMEM((K,), jnp.int32),
        pltpu.VMEM((K,), jnp.int32),
        pltpu.VMEM((K, D), jnp.float32),
        pltpu.VMEM_SHARED((NP, D), jnp.float32),
    ],
)
def _sc_scatter(ei_hbm, y_hbm, zeros_hbm, out_hbm, idx_src, idx_dst, rows, acc):
    c = lax.axis_index("c")
    s = lax.axis_index("s")
    wid = s * NC + c
    pltpu.sync_copy(zeros_hbm.at[pl.ds(s * RPS, RPS)], acc.at[pl.ds(s * RPS, RPS)])
    plsc.subcore_barrier()

    def chunk_body(g, _):
        base = (wid * CPT + g) * K
        pltpu.sync_copy(ei_hbm.at[0, pl.ds(base, K)], idx_src)
        pltpu.sync_copy(ei_hbm.at[1, pl.ds(base, K)], idx_dst)
        pltpu.sync_copy(y_hbm.at[idx_src], rows)
        pltpu.sync_copy(rows, acc.at[idx_dst], add=True)
        return ()

    lax.fori_loop(0, CPT, chunk_body, ())
    plsc.subcore_barrier()
    pltpu.sync_copy(acc.at[pl.ds(s * RPS, RPS)], out_hbm.at[c, pl.ds(s * RPS, RPS)])


@functools.partial(
    pl.kernel,
    out_type=jax.ShapeDtypeStruct((NC, NP, 16), jnp.float32),
    mesh=_mesh,
    scratch_types=[
        pltpu.VMEM((K,), jnp.int32),
        pltpu.VMEM((K, 16), jnp.float32),
        pltpu.VMEM_SHARED((NP, 16), jnp.float32),
    ],
)
def _sc_degree(ei_hbm, ones_hbm, zeros_hbm, out_hbm, idx_dst, ones_v, acc):
    c = lax.axis_index("c")
    s = lax.axis_index("s")
    wid = s * NC + c
    pltpu.sync_copy(ones_hbm, ones_v)
    pltpu.sync_copy(zeros_hbm.at[pl.ds(s * RPS, RPS)], acc.at[pl.ds(s * RPS, RPS)])
    plsc.subcore_barrier()

    def chunk_body(g, _):
        pltpu.sync_copy(ei_hbm.at[1, pl.ds((wid * CPT + g) * K, K)], idx_dst)
        pltpu.sync_copy(ones_v, acc.at[idx_dst], add=True)
        return ()

    lax.fori_loop(0, CPT, chunk_body, ())
    plsc.subcore_barrier()
    pltpu.sync_copy(acc.at[pl.ds(s * RPS, RPS)], out_hbm.at[c, pl.ds(s * RPS, RPS)])


_R = 2000


def _norm_block(degp_ref):
    deg = (degp_ref[0, :, 0].astype(jnp.float32)
           + degp_ref[1, :, 0].astype(jnp.float32))
    return lax.rsqrt(jnp.maximum(deg, 1.0))[:, None]


def _agg_block(aggp_ref):
    return aggp_ref[0].astype(jnp.float32) + aggp_ref[1].astype(jnp.float32)


def _degp_spec():
    return pl.BlockSpec((NC, _R, 16), lambda i: (0, i, 0))


def _row_spec():
    return pl.BlockSpec((_R, D), lambda i: (i, 0))


def _aggp_spec():
    return pl.BlockSpec((NC, _R, D), lambda i: (0, i, 0))


def _full_spec(r, c):
    return pl.BlockSpec((r, c), lambda i: (0, 0))


def _tc1(degp, xa, W1):
    def body(degp_ref, xa_ref, w_ref, y_ref):
        norm = _norm_block(degp_ref)
        xw = jnp.dot(xa_ref[...], w_ref[...], preferred_element_type=jnp.float32)
        y_ref[...] = xw * norm

    return pl.pallas_call(
        body,
        grid=(N // _R,),
        in_specs=[_degp_spec(), _row_spec(), _full_spec(D, D)],
        out_specs=_row_spec(),
        out_shape=jax.ShapeDtypeStruct((N, D), jnp.float32),
    )(degp, xa, W1)


def _tc2(degp, agg1p, b1, Wm, Ws):
    def body(degp_ref, aggp_ref, b_ref, wm_ref, ws_ref, ym_ref, ys_ref):
        norm = _norm_block(degp_ref)
        h = jnp.maximum(_agg_block(aggp_ref) * norm + b_ref[...], 0.0)
        ym = jnp.dot(h, wm_ref[...], preferred_element_type=jnp.float32) * norm
        ys = jnp.dot(h, ws_ref[...], preferred_element_type=jnp.float32) * norm
        ym_ref[...] = ym
        ys_ref[...] = ys

    return pl.pallas_call(
        body,
        grid=(N // _R,),
        in_specs=[_degp_spec(), _aggp_spec(), _full_spec(1, D),
                  _full_spec(D, D), _full_spec(D, D)],
        out_specs=[_row_spec(), _row_spec()],
        out_shape=[jax.ShapeDtypeStruct((N, D), jnp.float32),
                   jax.ShapeDtypeStruct((N, D), jnp.float32)],
    )(degp, agg1p, b1, Wm, Ws)


def _tc3(degp, aggmp, aggsp, bm, bs, noise, Wd):
    def body(degp_ref, aggmp_ref, aggsp_ref, bm_ref, bs_ref, noise_ref, wd_ref,
             yd_ref):
        norm = _norm_block(degp_ref)
        mean = _agg_block(aggmp_ref) * norm + bm_ref[...]
        logstd = _agg_block(aggsp_ref) * norm + bs_ref[...]
        z = noise_ref[...] * jnp.exp(logstd) + mean
        yd = jnp.dot(z, wd_ref[...], preferred_element_type=jnp.float32) * norm
        yd_ref[...] = yd

    return pl.pallas_call(
        body,
        grid=(N // _R,),
        in_specs=[_degp_spec(), _aggp_spec(), _aggp_spec(), _full_spec(1, D),
                  _full_spec(1, D), _row_spec(), _full_spec(D, D)],
        out_specs=_row_spec(),
        out_shape=jax.ShapeDtypeStruct((N, D), jnp.float32),
    )(degp, aggmp, aggsp, bm, bs, noise, Wd)


def _tc4(degp, aggdp, bd, Wl, bl):
    def body(degp_ref, aggp_ref, bd_ref, wl_ref, bl_ref, out_ref):
        norm = _norm_block(degp_ref)
        hd = jnp.maximum(_agg_block(aggp_ref) * norm + bd_ref[...], 0.0)
        out_ref[...] = (jnp.dot(hd, wl_ref[...], preferred_element_type=jnp.float32)
                        + bl_ref[...])

    return pl.pallas_call(
        body,
        grid=(N // _R,),
        in_specs=[_degp_spec(), _aggp_spec(), _full_spec(1, D),
                  _full_spec(D, D), _full_spec(1, D)],
        out_specs=_row_spec(),
        out_shape=jax.ShapeDtypeStruct((N, D), jnp.float32),
    )(degp, aggdp, bd, Wl, bl)


def kernel(xa, xb, edge_index, noise, W1, b1, Wm, bm, Ws, bs, Wd, bd, Wl, bl):
    del xb
    zeros128 = jnp.zeros((NP, D), jnp.float32)
    zeros16 = jnp.zeros((NP, 16), jnp.float32)
    ones16 = jnp.ones((K, 16), jnp.float32)
    pad = jnp.stack([jnp.zeros((EPAD,), jnp.int32),
                     jnp.full((EPAD,), PAD_NODE, jnp.int32)])
    ei3 = jnp.concatenate([edge_index, pad], axis=1)

    degp = _sc_degree(ei3, ones16, zeros16)
    y1 = _tc1(degp, xa, W1)
    agg1p = _sc_scatter(ei3, y1, zeros128)
    ym, ys = _tc2(degp, agg1p, b1.reshape(1, D), Wm, Ws)
    aggmp = _sc_scatter(ei3, ym, zeros128)
    aggsp = _sc_scatter(ei3, ys, zeros128)
    yd = _tc3(degp, aggmp, aggsp, bm.reshape(1, D), bs.reshape(1, D), noise, Wd)
    aggdp = _sc_scatter(ei3, yd, zeros128)
    return _tc4(degp, aggdp, bd.reshape(1, D), Wl, bl.reshape(1, D))


# speedup vs baseline: 1.2679x; 1.2679x over previous
import functools

import jax
import jax.numpy as jnp
from jax import lax
from jax.experimental import pallas as pl
from jax.experimental.pallas import tpu as pltpu
from jax.experimental.pallas import tpu_sc as plsc

N = 10000
E = 320000
D = 128

NC = 2
NS = 16
NW = NC * NS

K = 128
NCHUNK = 2560
CPT = NCHUNK // NW
EPAD = NCHUNK * K - E
NP = 10240
PAD_NODE = N + 100
RPS = NP // NS

_mesh = plsc.VectorSubcoreMesh(core_axis_name="c", subcore_axis_name="s")


@functools.partial(
    pl.kernel,
    out_type=jax.ShapeDtypeStruct((NC, NP, D), jnp.float32),
    mesh=_mesh,
    scratch_types=[
        pltpu.VMEM((K,), jnp.int32),
        pltpu.VMEM((K,), jnp.int32),
        pltpu.VMEM((K, D), jnp.float32),
        pltpu.VMEM_SHARED((NP, D), jnp.float32),
    ],
)
def _sc_scatter(ei_hbm, y_hbm, zeros_hbm, out_hbm, idx_src, idx_dst, rows, acc):
    c = lax.axis_index("c")
    s = lax.axis_index("s")
    wid = s * NC + c
    pltpu.sync_copy(zeros_hbm.at[pl.ds(s * RPS, RPS)], acc.at[pl.ds(s * RPS, RPS)])
    plsc.subcore_barrier()

    def chunk_body(g, _):
        base = (wid + g * NW) * K
        pltpu.sync_copy(ei_hbm.at[0, pl.ds(base, K)], idx_src)
        pltpu.sync_copy(ei_hbm.at[1, pl.ds(base, K)], idx_dst)
        pltpu.sync_copy(y_hbm.at[idx_src], rows)
        pltpu.sync_copy(rows, acc.at[idx_dst], add=True)
        return ()

    lax.fori_loop(0, CPT, chunk_body, ())
    plsc.subcore_barrier()
    pltpu.sync_copy(acc.at[pl.ds(s * RPS, RPS)], out_hbm.at[c, pl.ds(s * RPS, RPS)])


CPT2 = NCHUNK // NS


@functools.partial(
    pl.kernel,
    out_type=jax.ShapeDtypeStruct((NC, NP, D), jnp.float32),
    mesh=_mesh,
    scratch_types=[
        pltpu.VMEM((K,), jnp.int32),
        pltpu.VMEM((K,), jnp.int32),
        pltpu.VMEM((K, D), jnp.float32),
        pltpu.VMEM_SHARED((NP, D), jnp.float32),
    ],
)
def _sc_scatter2(ei_hbm, ym_hbm, ys_hbm, zeros_hbm, out_hbm, idx_src, idx_dst,
                 rows, acc):
    c = lax.axis_index("c")
    s = lax.axis_index("s")
    pltpu.sync_copy(zeros_hbm.at[pl.ds(s * RPS, RPS)], acc.at[pl.ds(s * RPS, RPS)])
    plsc.subcore_barrier()

    def make_body(y_hbm):
        def chunk_body(g, _):
            base = (s + g * NS) * K
            pltpu.sync_copy(ei_hbm.at[0, pl.ds(base, K)], idx_src)
            pltpu.sync_copy(ei_hbm.at[1, pl.ds(base, K)], idx_dst)
            pltpu.sync_copy(y_hbm.at[idx_src], rows)
            pltpu.sync_copy(rows, acc.at[idx_dst], add=True)
            return ()

        return chunk_body

    @pl.when(c == 0)
    def _():
        lax.fori_loop(0, CPT2, make_body(ym_hbm), ())

    @pl.when(c == 1)
    def _():
        lax.fori_loop(0, CPT2, make_body(ys_hbm), ())

    plsc.subcore_barrier()
    pltpu.sync_copy(acc.at[pl.ds(s * RPS, RPS)], out_hbm.at[c, pl.ds(s * RPS, RPS)])


@functools.partial(
    pl.kernel,
    out_type=jax.ShapeDtypeStruct((NC, NP, 16), jnp.float32),
    mesh=_mesh,
    scratch_types=[
        pltpu.VMEM((K,), jnp.int32),
        pltpu.VMEM((K, 16), jnp.float32),
        pltpu.VMEM_SHARED((NP, 16), jnp.float32),
    ],
)
def _sc_degree(ei_hbm, ones_hbm, zeros_hbm, out_hbm, idx_dst, ones_v, acc):
    c = lax.axis_index("c")
    s = lax.axis_index("s")
    wid = s * NC + c
    pltpu.sync_copy(ones_hbm, ones_v)
    pltpu.sync_copy(zeros_hbm.at[pl.ds(s * RPS, RPS)], acc.at[pl.ds(s * RPS, RPS)])
    plsc.subcore_barrier()

    def chunk_body(g, _):
        pltpu.sync_copy(ei_hbm.at[1, pl.ds((wid + g * NW) * K, K)], idx_dst)
        pltpu.sync_copy(ones_v, acc.at[idx_dst], add=True)
        return ()

    lax.fori_loop(0, CPT, chunk_body, ())
    plsc.subcore_barrier()
    pltpu.sync_copy(acc.at[pl.ds(s * RPS, RPS)], out_hbm.at[c, pl.ds(s * RPS, RPS)])


_R = 2000


def _norm_block(degp_ref):
    deg = (degp_ref[0, :, 0].astype(jnp.float32)
           + degp_ref[1, :, 0].astype(jnp.float32))
    return lax.rsqrt(jnp.maximum(deg, 1.0))[:, None]


def _agg_block(aggp_ref):
    return aggp_ref[0].astype(jnp.float32) + aggp_ref[1].astype(jnp.float32)


def _degp_spec():
    return pl.BlockSpec((NC, _R, 16), lambda i: (0, i, 0))


def _row_spec():
    return pl.BlockSpec((_R, D), lambda i: (i, 0))


def _aggp_spec():
    return pl.BlockSpec((NC, _R, D), lambda i: (0, i, 0))


def _full_spec(r, c):
    return pl.BlockSpec((r, c), lambda i: (0, 0))


def _tc1(degp, xa, W1):
    def body(degp_ref, xa_ref, w_ref, y_ref):
        norm = _norm_block(degp_ref)
        xw = jnp.dot(xa_ref[...], w_ref[...], preferred_element_type=jnp.float32)
        y_ref[...] = xw * norm

    return pl.pallas_call(
        body,
        grid=(N // _R,),
        in_specs=[_degp_spec(), _row_spec(), _full_spec(D, D)],
        out_specs=_row_spec(),
        out_shape=jax.ShapeDtypeStruct((N, D), jnp.float32),
    )(degp, xa, W1)


def _tc2(degp, agg1p, b1, Wm, Ws):
    def body(degp_ref, aggp_ref, b_ref, wm_ref, ws_ref, ym_ref, ys_ref):
        norm = _norm_block(degp_ref)
        h = jnp.maximum(_agg_block(aggp_ref) * norm + b_ref[...], 0.0)
        ym = jnp.dot(h, wm_ref[...], preferred_element_type=jnp.float32) * norm
        ys = jnp.dot(h, ws_ref[...], preferred_element_type=jnp.float32) * norm
        ym_ref[...] = ym
        ys_ref[...] = ys

    return pl.pallas_call(
        body,
        grid=(N // _R,),
        in_specs=[_degp_spec(), _aggp_spec(), _full_spec(1, D),
                  _full_spec(D, D), _full_spec(D, D)],
        out_specs=[_row_spec(), _row_spec()],
        out_shape=[jax.ShapeDtypeStruct((N, D), jnp.float32),
                   jax.ShapeDtypeStruct((N, D), jnp.float32)],
    )(degp, agg1p, b1, Wm, Ws)


def _tc3(degp, aggms, bm, bs, noise, Wd):
    def body(degp_ref, aggms_ref, bm_ref, bs_ref, noise_ref, wd_ref, yd_ref):
        norm = _norm_block(degp_ref)
        mean = aggms_ref[0] * norm + bm_ref[...]
        logstd = aggms_ref[1] * norm + bs_ref[...]
        z = noise_ref[...] * jnp.exp(logstd) + mean
        yd = jnp.dot(z, wd_ref[...], preferred_element_type=jnp.float32) * norm
        yd_ref[...] = yd

    return pl.pallas_call(
        body,
        grid=(N // _R,),
        in_specs=[_degp_spec(), _aggp_spec(), _full_spec(1, D),
                  _full_spec(1, D), _row_spec(), _full_spec(D, D)],
        out_specs=_row_spec(),
        out_shape=jax.ShapeDtypeStruct((N, D), jnp.float32),
    )(degp, aggms, bm, bs, noise, Wd)


def _tc4(degp, aggdp, bd, Wl, bl):
    def body(degp_ref, aggp_ref, bd_ref, wl_ref, bl_ref, out_ref):
        norm = _norm_block(degp_ref)
        hd = jnp.maximum(_agg_block(aggp_ref) * norm + bd_ref[...], 0.0)
        out_ref[...] = (jnp.dot(hd, wl_ref[...], preferred_element_type=jnp.float32)
                        + bl_ref[...])

    return pl.pallas_call(
        body,
        grid=(N // _R,),
        in_specs=[_degp_spec(), _aggp_spec(), _full_spec(1, D),
                  _full_spec(D, D), _full_spec(1, D)],
        out_specs=_row_spec(),
        out_shape=jax.ShapeDtypeStruct((N, D), jnp.float32),
    )(degp, aggdp, bd, Wl, bl)


def kernel(xa, xb, edge_index, noise, W1, b1, Wm, bm, Ws, bs, Wd, bd, Wl, bl):
    del xb
    zeros128 = jnp.zeros((NP, D), jnp.float32)
    zeros16 = jnp.zeros((NP, 16), jnp.float32)
    ones16 = jnp.ones((K, 16), jnp.float32)
    pad = jnp.stack([jnp.zeros((EPAD,), jnp.int32),
                     jnp.full((EPAD,), PAD_NODE, jnp.int32)])
    ei3 = jnp.concatenate([edge_index, pad], axis=1)

    degp = _sc_degree(ei3, ones16, zeros16)
    y1 = _tc1(degp, xa, W1)
    agg1p = _sc_scatter(ei3, y1, zeros128)
    ym, ys = _tc2(degp, agg1p, b1.reshape(1, D), Wm, Ws)
    aggms = _sc_scatter2(ei3, ym, ys, zeros128)
    yd = _tc3(degp, aggms, bm.reshape(1, D), bs.reshape(1, D), noise, Wd)
    aggdp = _sc_scatter(ei3, yd, zeros128)
    return _tc4(degp, aggdp, bd.reshape(1, D), Wl, bl.reshape(1, D))


# speedup vs baseline: 2.1868x; 1.7248x over previous
import functools

import jax
import jax.numpy as jnp
from jax import lax
from jax.experimental import pallas as pl
from jax.experimental.pallas import tpu as pltpu
from jax.experimental.pallas import tpu_sc as plsc

N = 10000
E = 320000
D = 128

NC = 2
NS = 16
NW = NC * NS

K = 128
NCHUNK = E // K
CPT = -(-NCHUNK // NW)
NP = 10240
RPS = NP // NS

_mesh = plsc.VectorSubcoreMesh(core_axis_name="c", subcore_axis_name="s")


@functools.partial(
    pl.kernel,
    out_type=jax.ShapeDtypeStruct((NC, NP, D), jnp.float32),
    mesh=_mesh,
    scratch_types=[
        pltpu.VMEM((K,), jnp.int32),
        pltpu.VMEM((K,), jnp.int32),
        pltpu.VMEM((K, D), jnp.float32),
        pltpu.VMEM_SHARED((NP, D), jnp.float32),
    ],
)
def _sc_scatter(ei_hbm, y_hbm, zeros_hbm, out_hbm, idx_src, idx_dst, rows, acc):
    c = lax.axis_index("c")
    s = lax.axis_index("s")
    wid = s * NC + c
    pltpu.sync_copy(zeros_hbm.at[pl.ds(s * RPS, RPS)], acc.at[pl.ds(s * RPS, RPS)])
    plsc.subcore_barrier()

    def chunk_body(g, _):
        chunk = wid + g * NW

        @pl.when(chunk < NCHUNK)
        def _():
            base = chunk * K
            pltpu.sync_copy(ei_hbm.at[0, pl.ds(base, K)], idx_src)
            pltpu.sync_copy(ei_hbm.at[1, pl.ds(base, K)], idx_dst)
            pltpu.sync_copy(y_hbm.at[idx_src], rows)
            pltpu.sync_copy(rows, acc.at[idx_dst], add=True)

        return ()

    lax.fori_loop(0, CPT, chunk_body, ())
    plsc.subcore_barrier()
    pltpu.sync_copy(acc.at[pl.ds(s * RPS, RPS)], out_hbm.at[c, pl.ds(s * RPS, RPS)])


@functools.partial(
    pl.kernel,
    out_type=jax.ShapeDtypeStruct((NC, NP, 16), jnp.float32),
    mesh=_mesh,
    scratch_types=[
        pltpu.VMEM((K,), jnp.int32),
        pltpu.VMEM((K, 16), jnp.float32),
        pltpu.VMEM_SHARED((NP, 16), jnp.float32),
    ],
)
def _sc_degree(ei_hbm, ones_hbm, zeros_hbm, out_hbm, idx_dst, ones_v, acc):
    c = lax.axis_index("c")
    s = lax.axis_index("s")
    wid = s * NC + c
    pltpu.sync_copy(ones_hbm, ones_v)
    pltpu.sync_copy(zeros_hbm.at[pl.ds(s * RPS, RPS)], acc.at[pl.ds(s * RPS, RPS)])
    plsc.subcore_barrier()

    def chunk_body(g, _):
        chunk = wid + g * NW

        @pl.when(chunk < NCHUNK)
        def _():
            pltpu.sync_copy(ei_hbm.at[1, pl.ds(chunk * K, K)], idx_dst)
            pltpu.sync_copy(ones_v, acc.at[idx_dst], add=True)

        return ()

    lax.fori_loop(0, CPT, chunk_body, ())
    plsc.subcore_barrier()
    pltpu.sync_copy(acc.at[pl.ds(s * RPS, RPS)], out_hbm.at[c, pl.ds(s * RPS, RPS)])


_R = 2000


def _norm_block(degp_ref):
    deg = (degp_ref[0, :, 0].astype(jnp.float32)
           + degp_ref[1, :, 0].astype(jnp.float32))
    return lax.rsqrt(jnp.maximum(deg, 1.0))[:, None]


def _agg_block(aggp_ref):
    return aggp_ref[0].astype(jnp.float32) + aggp_ref[1].astype(jnp.float32)


def _degp_spec():
    return pl.BlockSpec((NC, _R, 16), lambda i: (0, i, 0))


def _row_spec():
    return pl.BlockSpec((_R, D), lambda i: (i, 0))


def _aggp_spec():
    return pl.BlockSpec((NC, _R, D), lambda i: (0, i, 0))


def _full_spec(r, c):
    return pl.BlockSpec((r, c), lambda i: (0, 0))


def _tc1(degp, xa, W1):
    def body(degp_ref, xa_ref, w_ref, y_ref):
        norm = _norm_block(degp_ref)
        xw = jnp.dot(xa_ref[...], w_ref[...], preferred_element_type=jnp.float32)
        y_ref[...] = xw * norm

    return pl.pallas_call(
        body,
        grid=(N // _R,),
        in_specs=[_degp_spec(), _row_spec(), _full_spec(D, D)],
        out_specs=_row_spec(),
        out_shape=jax.ShapeDtypeStruct((N, D), jnp.float32),
    )(degp, xa, W1)


def _tc2(degp, agg1p, b1, Wm, Ws):
    def body(degp_ref, aggp_ref, b_ref, wm_ref, ws_ref, ym_ref, ys_ref):
        norm = _norm_block(degp_ref)
        h = jnp.maximum(_agg_block(aggp_ref) * norm + b_ref[...], 0.0)
        ym = jnp.dot(h, wm_ref[...], preferred_element_type=jnp.float32) * norm
        ys = jnp.dot(h, ws_ref[...], preferred_element_type=jnp.float32) * norm
        ym_ref[...] = ym
        ys_ref[...] = ys

    return pl.pallas_call(
        body,
        grid=(N // _R,),
        in_specs=[_degp_spec(), _aggp_spec(), _full_spec(1, D),
                  _full_spec(D, D), _full_spec(D, D)],
        out_specs=[_row_spec(), _row_spec()],
        out_shape=[jax.ShapeDtypeStruct((N, D), jnp.float32),
                   jax.ShapeDtypeStruct((N, D), jnp.float32)],
    )(degp, agg1p, b1, Wm, Ws)


def _tc3(degp, aggmp, aggsp, bm, bs, noise, Wd):
    def body(degp_ref, aggmp_ref, aggsp_ref, bm_ref, bs_ref, noise_ref, wd_ref,
             yd_ref):
        norm = _norm_block(degp_ref)
        mean = _agg_block(aggmp_ref) * norm + bm_ref[...]
        logstd = _agg_block(aggsp_ref) * norm + bs_ref[...]
        z = noise_ref[...] * jnp.exp(logstd) + mean
        yd = jnp.dot(z, wd_ref[...], preferred_element_type=jnp.float32) * norm
        yd_ref[...] = yd

    return pl.pallas_call(
        body,
        grid=(N // _R,),
        in_specs=[_degp_spec(), _aggp_spec(), _aggp_spec(), _full_spec(1, D),
                  _full_spec(1, D), _row_spec(), _full_spec(D, D)],
        out_specs=_row_spec(),
        out_shape=jax.ShapeDtypeStruct((N, D), jnp.float32),
    )(degp, aggmp, aggsp, bm, bs, noise, Wd)


def _tc4(degp, aggdp, bd, Wl, bl):
    def body(degp_ref, aggp_ref, bd_ref, wl_ref, bl_ref, out_ref):
        norm = _norm_block(degp_ref)
        hd = jnp.maximum(_agg_block(aggp_ref) * norm + bd_ref[...], 0.0)
        out_ref[...] = (jnp.dot(hd, wl_ref[...], preferred_element_type=jnp.float32)
                        + bl_ref[...])

    return pl.pallas_call(
        body,
        grid=(N // _R,),
        in_specs=[_degp_spec(), _aggp_spec(), _full_spec(1, D),
                  _full_spec(D, D), _full_spec(1, D)],
        out_specs=_row_spec(),
        out_shape=jax.ShapeDtypeStruct((N, D), jnp.float32),
    )(degp, aggdp, bd, Wl, bl)


def kernel(xa, xb, edge_index, noise, W1, b1, Wm, bm, Ws, bs, Wd, bd, Wl, bl):
    del xb
    zeros128 = jnp.zeros((NP, D), jnp.float32)
    zeros16 = jnp.zeros((NP, 16), jnp.float32)
    ones16 = jnp.ones((K, 16), jnp.float32)

    degp = _sc_degree(edge_index, ones16, zeros16)
    y1 = _tc1(degp, xa, W1)
    agg1p = _sc_scatter(edge_index, y1, zeros128)
    ym, ys = _tc2(degp, agg1p, b1.reshape(1, D), Wm, Ws)
    aggmp = _sc_scatter(edge_index, ym, zeros128)
    aggsp = _sc_scatter(edge_index, ys, zeros128)
    yd = _tc3(degp, aggmp, aggsp, bm.reshape(1, D), bs.reshape(1, D), noise, Wd)
    aggdp = _sc_scatter(edge_index, yd, zeros128)
    return _tc4(degp, aggdp, bd.reshape(1, D), Wl, bl.reshape(1, D))
